# baseline (speedup 1.0000x reference)
import jax
import jax.numpy as jnp
from jax.experimental import pallas as pl
from jax.experimental.pallas import tpu as pltpu

B, N, HD, NFD, EFD, TD, L = 16, 64, 128, 6, 15, 4, 3
_BN_SCALE = float(1.0 / (1.0 + 1e-5) ** 0.5)
_F32 = jnp.float32


def _dot(a, b):
    return jnp.dot(a, b, preferred_element_type=_F32)


def _sgcn_kernel(ef_ref, tf_ref, nf_ref, adj_ref,
                 embW_ref, embb_ref, nbg_ref, nbb_ref,
                 Wh_ref, We_ref, mb_ref,
                 Wue_ref, Wum_ref, ub_ref, bng_ref, bnb_ref,
                 W1a_ref, W1b_ref, W1c_ref, W1d_ref, b1_ref,
                 l1g_ref, l1b_ref,
                 W2_ref, b2_ref, l2g_ref, l2b_ref,
                 w3_ref, b3_ref,
                 out_ref):
    nf = nf_ref[0]
    emb = _dot(nf, embW_ref[...].T) + embb_ref[...]
    emb = emb * (_BN_SCALE * nbg_ref[...]) + nbb_ref[...]
    emb = jnp.maximum(emb, 0.0)

    adj = adj_ref[...]
    mask = (adj > 0).astype(_F32)
    deg = jnp.sum(mask, axis=1, keepdims=True)

    ef = ef_ref[0]
    er3 = ef[:, :, EFD - 3:]
    er_sum = jnp.sum(er3 * mask.T[:, :, None], axis=0)

    for l in range(L):
        A = _dot(emb, Wh_ref[l].T)
        msgs = _dot(mask, A)
        msgs = msgs + _dot(er_sum, We_ref[l].T)
        msgs = msgs + deg * mb_ref[l][None, :]
        upd = _dot(emb, Wue_ref[l].T) + _dot(msgs, Wum_ref[l].T) + ub_ref[l][None, :]
        upd = jnp.maximum(upd, 0.0)
        upd = upd * (_BN_SCALE * bng_ref[l][None, :]) + bnb_ref[l][None, :]
        emb = upd + emb

    P = _dot(emb, W1a_ref[...].T) + b1_ref[...]
    Q = _dot(emb, W1b_ref[...].T)
    er2 = er3.reshape(N * N, 3)
    t2 = tf_ref[0].reshape(N * N, TD)
    R = _dot(er2, W1c_ref[...].T) + _dot(t2, W1d_ref[...].T)
    h = R.reshape(N, N, 128) + P[:, None, :] + Q[None, :, :]
    h = h.reshape(N * N, 128)
    O1 = jnp.full((128, 128), 1.0 / 128, _F32)
    h = h - _dot(h, O1)
    v = _dot(h * h, O1)
    h = h * jax.lax.rsqrt(v + 1e-5) * l1g_ref[...] + l1b_ref[...]
    h = jnp.maximum(h, 0.0)

    h = _dot(h, W2_ref[...].T) + b2_ref[...]
    O2 = jnp.full((64, 64), 1.0 / 64, _F32)
    h = h - _dot(h, O2)
    v = _dot(h * h, O2)
    h = h * jax.lax.rsqrt(v + 1e-5) * l2g_ref[...] + l2b_ref[...]
    h = jnp.maximum(h, 0.0)

    out = _dot(h, w3_ref[...])[:, :1] + b3_ref[0, 0]
    out_ref[0] = jnp.maximum(out, 0.0)


def kernel(node_features, edge_features, temporal_features, adjacency,
           emb_W, emb_b, node_bn_g, node_bn_b,
           msg_W, msg_b, upd_W, upd_b, bn_g, bn_b,
           fp_W1, fp_b1, ln1_g, ln1_b,
           fp_W2, fp_b2, ln2_g, ln2_b,
           fp_W3, fp_b3):
    Wh = msg_W[:, :, :HD]
    We = msg_W[:, :, HD:]
    Wue = upd_W[:, :, :HD]
    Wum = upd_W[:, :, HD:]
    W1a = fp_W1[:, :HD]
    W1b = fp_W1[:, HD:2 * HD]
    W1c = fp_W1[:, 2 * HD:2 * HD + 3]
    W1d = fp_W1[:, 2 * HD + 3:]
    w3_rep = jnp.repeat(fp_W3.reshape(64, 1), 128, axis=1)
    row = lambda x: x.reshape(1, -1)

    full = lambda shape: pl.BlockSpec(shape, lambda b: (0,) * len(shape))
    out = pl.pallas_call(
        _sgcn_kernel,
        grid=(B,),
        in_specs=[
            pl.BlockSpec((1, N, N, EFD), lambda b: (b, 0, 0, 0)),
            pl.BlockSpec((1, N, N, TD), lambda b: (b, 0, 0, 0)),
            pl.BlockSpec((1, N, NFD), lambda b: (b, 0, 0)),
            full((N, N)),
            full((HD, NFD)), full((1, HD)), full((1, HD)), full((1, HD)),
            full((L, HD, HD)), full((L, HD, 3)), full((L, HD)),
            full((L, HD, HD)), full((L, HD, HD)), full((L, HD)),
            full((L, HD)), full((L, HD)),
            full((128, HD)), full((128, HD)), full((128, 3)), full((128, TD)),
            full((1, 128)),
            full((1, 128)), full((1, 128)),
            full((64, 128)), full((1, 64)), full((1, 64)), full((1, 64)),
            full((64, 128)), full((1, 1)),
        ],
        out_specs=pl.BlockSpec((1, N * N, 1), lambda b: (b, 0, 0)),
        out_shape=jax.ShapeDtypeStruct((B, N * N, 1), _F32),
        compiler_params=pltpu.CompilerParams(
            dimension_semantics=("parallel",),
        ),
    )(edge_features, temporal_features, node_features, adjacency,
      emb_W, row(emb_b), row(node_bn_g), row(node_bn_b),
      Wh, We, msg_b,
      Wue, Wum, upd_b, bn_g, bn_b,
      W1a, W1b, W1c, W1d, row(fp_b1),
      row(ln1_g), row(ln1_b),
      fp_W2, row(fp_b2), row(ln2_g), row(ln2_b),
      w3_rep, row(fp_b3).reshape(1, 1))
    return out.reshape(B, N, N)


# speedup vs baseline: 1.3152x; 1.3152x over previous
import jax
import jax.numpy as jnp
from jax.experimental import pallas as pl
from jax.experimental.pallas import tpu as pltpu

B, N, HD, NFD, EFD, TD, L = 16, 64, 128, 6, 15, 4, 3
BB = 4
_BN_SCALE = float(1.0 / (1.0 + 1e-5) ** 0.5)
_F32 = jnp.float32


def _dot(a, b):
    return jnp.dot(a, b, preferred_element_type=_F32)


def _sgcn_kernel(et_ref, nf_ref, adj_ref,
                 embW_ref, embb_ref, nbg_ref, nbb_ref,
                 Wh_ref, We_ref, mb_ref,
                 Wue_ref, Wum_ref, ub_ref, bng_ref, bnb_ref,
                 W1a_ref, W1b_ref, W1cd_ref, b1_ref,
                 l1g_ref, l1b_ref,
                 W2_ref, b2_ref, l2g_ref, l2b_ref,
                 w3_ref, b3_ref,
                 out_ref):
    adj = adj_ref[...]
    mask = (adj > 0).astype(_F32)
    maskT = mask.T
    deg = jnp.sum(mask, axis=1, keepdims=True)
    O1 = jnp.full((128, 128), 1.0 / 128, _F32)
    O2 = jnp.full((64, 64), 1.0 / 64, _F32)

    for bb in range(BB):
        nf = nf_ref[bb]
        emb = _dot(nf, embW_ref[...].T) + embb_ref[...]
        emb = emb * (_BN_SCALE * nbg_ref[...]) + nbb_ref[...]
        emb = jnp.maximum(emb, 0.0)

        et = et_ref[bb]
        er3 = et[:, :3].reshape(N, N, 3)
        er_sum = jnp.sum(er3 * maskT[:, :, None], axis=0)

        for l in range(L):
            A = _dot(emb, Wh_ref[l].T)
            msgs = _dot(mask, A)
            msgs = msgs + _dot(er_sum, We_ref[l].T)
            msgs = msgs + deg * mb_ref[l][None, :]
            upd = _dot(emb, Wue_ref[l].T) + _dot(msgs, Wum_ref[l].T) + ub_ref[l][None, :]
            upd = jnp.maximum(upd, 0.0)
            upd = upd * (_BN_SCALE * bng_ref[l][None, :]) + bnb_ref[l][None, :]
            emb = upd + emb

        P = _dot(emb, W1a_ref[...].T) + b1_ref[...]
        Q = _dot(emb, W1b_ref[...].T)
        R = _dot(et, W1cd_ref[...].T)
        h = R.reshape(N, N, 128) + P[:, None, :] + Q[None, :, :]
        h = h.reshape(N * N, 128)
        h = h - _dot(h, O1)
        v = _dot(h * h, O1)
        h = h * jax.lax.rsqrt(v + 1e-5) * l1g_ref[...] + l1b_ref[...]
        h = jnp.maximum(h, 0.0)

        h = _dot(h, W2_ref[...].T) + b2_ref[...]
        h = h - _dot(h, O2)
        v = _dot(h * h, O2)
        h = h * jax.lax.rsqrt(v + 1e-5) * l2g_ref[...] + l2b_ref[...]
        h = jnp.maximum(h, 0.0)

        out = _dot(h, w3_ref[...])[:, :1] + b3_ref[0, 0]
        out_ref[bb] = jnp.maximum(out, 0.0)


def kernel(node_features, edge_features, temporal_features, adjacency,
           emb_W, emb_b, node_bn_g, node_bn_b,
           msg_W, msg_b, upd_W, upd_b, bn_g, bn_b,
           fp_W1, fp_b1, ln1_g, ln1_b,
           fp_W2, fp_b2, ln2_g, ln2_b,
           fp_W3, fp_b3):
    et = jnp.concatenate(
        [edge_features[..., EFD - 3:], temporal_features], axis=-1
    ).reshape(B, N * N, 3 + TD)
    Wh = msg_W[:, :, :HD]
    We = msg_W[:, :, HD:]
    Wue = upd_W[:, :, :HD]
    Wum = upd_W[:, :, HD:]
    W1a = fp_W1[:, :HD]
    W1b = fp_W1[:, HD:2 * HD]
    W1cd = fp_W1[:, 2 * HD:]
    w3_rep = jnp.repeat(fp_W3.reshape(64, 1), 128, axis=1)
    row = lambda x: x.reshape(1, -1)

    full = lambda shape: pl.BlockSpec(shape, lambda b: (0,) * len(shape))
    out = pl.pallas_call(
        _sgcn_kernel,
        grid=(B // BB,),
        in_specs=[
            pl.BlockSpec((BB, N * N, 3 + TD), lambda b: (b, 0, 0)),
            pl.BlockSpec((BB, N, NFD), lambda b: (b, 0, 0)),
            full((N, N)),
            full((HD, NFD)), full((1, HD)), full((1, HD)), full((1, HD)),
            full((L, HD, HD)), full((L, HD, 3)), full((L, HD)),
            full((L, HD, HD)), full((L, HD, HD)), full((L, HD)),
            full((L, HD)), full((L, HD)),
            full((128, HD)), full((128, HD)), full((128, 3 + TD)),
            full((1, 128)),
            full((1, 128)), full((1, 128)),
            full((64, 128)), full((1, 64)), full((1, 64)), full((1, 64)),
            full((64, 128)), full((1, 1)),
        ],
        out_specs=pl.BlockSpec((BB, N * N, 1), lambda b: (b, 0, 0)),
        out_shape=jax.ShapeDtypeStruct((B, N * N, 1), _F32),
        compiler_params=pltpu.CompilerParams(
            dimension_semantics=("arbitrary",),
        ),
    )(et, node_features, adjacency,
      emb_W, row(emb_b), row(node_bn_g), row(node_bn_b),
      Wh, We, msg_b,
      Wue, Wum, upd_b, bn_g, bn_b,
      W1a, W1b, W1cd, row(fp_b1),
      row(ln1_g), row(ln1_b),
      fp_W2, row(fp_b2), row(ln2_g), row(ln2_b),
      w3_rep, row(fp_b3).reshape(1, 1))
    return out.reshape(B, N, N)
